# baseline (device time: 623017 ns/iter reference)
import jax
import jax.numpy as jnp
from jax import lax
from jax.experimental import pallas as pl
from jax.experimental.pallas import tpu as pltpu

N_DEV = 4
M_PER = 2048
K_BLK = 2048
N_TOT = 4096
NTW = 256
NTS = N_TOT // NTW
N_STEPS = N_DEV * NTS


def kernel(x, w_mat):
    x16 = x.astype(jnp.bfloat16)

    def body(x_ref, w_ref, o_ref, xg, xb, wf, lsem, xbsem, wsems,
             send_sems, recv_sems):
        j = pl.program_id(0)
        nt = pl.program_id(1)
        s = j * NTS + nt
        me = lax.axis_index("i")

        def src_of(jj):
            return lax.rem(me + lax.bitwise_xor(jj, jj // 2), N_DEV)

        def mk_send(off, slot):
            t = lax.rem(me + off, N_DEV)
            return pltpu.make_async_remote_copy(
                src_ref=x_ref.at[pl.ds(t * M_PER, M_PER), :],
                dst_ref=xg.at[me],
                send_sem=send_sems.at[slot],
                recv_sem=recv_sems.at[me],
                device_id=(t,),
                device_id_type=pl.DeviceIdType.MESH,
            )

        def mk_recv(src):
            return pltpu.make_async_remote_copy(
                src_ref=xg.at[src],
                dst_ref=xg.at[src],
                send_sem=send_sems.at[0],
                recv_sem=recv_sems.at[src],
                device_id=(src,),
                device_id_type=pl.DeviceIdType.MESH,
            )

        def mk_xload(jj):
            return pltpu.make_async_copy(xg.at[src_of(jj)], xb, xbsem)

        def mk_wload(jj, tt, slot):
            return pltpu.make_async_copy(
                w_ref.at[pl.ds(src_of(jj) * K_BLK, K_BLK),
                         pl.ds(tt * NTW, NTW)],
                wf.at[slot],
                wsems.at[slot],
            )

        slot = lax.rem(s, 2)

        @pl.when(s == 0)
        def _():
            barrier_sem = pltpu.get_barrier_semaphore()
            for off in (1, 2, 3):
                t = lax.rem(me + off, N_DEV)
                pl.semaphore_signal(
                    barrier_sem, inc=1,
                    device_id=(t,), device_id_type=pl.DeviceIdType.MESH,
                )
            pl.semaphore_wait(barrier_sem, 3)
            mk_send(1, 0).start()
            mk_send(3, 1).start()
            mk_send(2, 2).start()
            cp = pltpu.make_async_copy(
                x_ref.at[pl.ds(me * M_PER, M_PER), :], xg.at[me], lsem
            )
            cp.start()
            cp.wait()
            mk_xload(0).start()
            mk_wload(0, 0, 0).start()

        @pl.when(nt == 0)
        def _():
            mk_xload(j).wait()

        mk_wload(j, nt, slot).wait()

        val = jnp.dot(
            xb[...],
            wf[slot].astype(jnp.bfloat16),
            preferred_element_type=jnp.float32,
        )
        cols = pl.ds(nt * NTW, NTW)

        @pl.when(j == 0)
        def _():
            o_ref[:, cols] = val

        @pl.when(j > 0)
        def _():
            o_ref[:, cols] += val

        @pl.when(s < N_STEPS - 1)
        def _():
            sn = s + 1
            jn = sn // NTS
            ntn = lax.rem(sn, NTS)
            mk_wload(jn, ntn, 1 - slot).start()

            @pl.when(ntn == 0)
            def _():
                mk_recv(src_of(jn)).wait_recv()
                mk_xload(jn).start()

        @pl.when(s == N_STEPS - 1)
        def _():
            for tt in range(NTS):
                c = slice(tt * NTW, (tt + 1) * NTW)
                o_ref[:, c] = jnp.maximum(o_ref[:, c], 0.0)
            for off, sl in ((1, 0), (3, 1), (2, 2)):
                mk_send(off, sl).wait_send()

    out, _ = pl.pallas_call(
        body,
        grid=(N_DEV, NTS),
        out_shape=[
            jax.ShapeDtypeStruct((M_PER, N_TOT), jnp.float32),
            jax.ShapeDtypeStruct((N_DEV, M_PER, K_BLK), jnp.bfloat16),
        ],
        in_specs=[
            pl.BlockSpec(memory_space=pl.ANY),
            pl.BlockSpec(memory_space=pl.ANY),
        ],
        out_specs=[
            pl.BlockSpec((M_PER, N_TOT), lambda j, nt: (0, 0)),
            pl.BlockSpec(memory_space=pl.ANY),
        ],
        scratch_shapes=[
            pltpu.VMEM((M_PER, K_BLK), jnp.bfloat16),
            pltpu.VMEM((2, K_BLK, NTW), jnp.float32),
            pltpu.SemaphoreType.DMA,
            pltpu.SemaphoreType.DMA,
            pltpu.SemaphoreType.DMA((2,)),
            pltpu.SemaphoreType.DMA((3,)),
            pltpu.SemaphoreType.DMA((N_DEV,)),
        ],
        compiler_params=pltpu.CompilerParams(
            collective_id=0,
            vmem_limit_bytes=60 * 1024 * 1024,
        ),
    )(x16, w_mat)
    return out


# device time: 539847 ns/iter; 1.1541x vs baseline; 1.1541x over previous
import jax
import jax.numpy as jnp
from jax import lax
from jax.experimental import pallas as pl
from jax.experimental.pallas import tpu as pltpu

N_DEV = 4
M_PER = 2048
K_BLK = 2048
N_TOT = 4096
NTW = 256
NTS = N_TOT // NTW
WFR = 256
WF = K_BLK // WFR
N_STEPS = N_DEV * NTS


def kernel(x, w_mat):
    x16 = x.astype(jnp.bfloat16)

    def body(x_ref, w_ref, o_ref, xg, o16, wblk, xb, wstage, fstage,
             lsem, xbsem, wsems, fsems, send_sems, recv_sems):
        j = pl.program_id(0)
        nt = pl.program_id(1)
        s = j * NTS + nt
        me = lax.axis_index("i")

        def src_of(jj):
            return lax.rem(me + lax.bitwise_xor(jj, jj // 2), N_DEV)

        def mk_send(off, slot):
            t = lax.rem(me + off, N_DEV)
            return pltpu.make_async_remote_copy(
                src_ref=x_ref.at[pl.ds(t * M_PER, M_PER), :],
                dst_ref=xg.at[me],
                send_sem=send_sems.at[slot],
                recv_sem=recv_sems.at[me],
                device_id=(t,),
                device_id_type=pl.DeviceIdType.MESH,
            )

        def mk_recv(src):
            return pltpu.make_async_remote_copy(
                src_ref=xg.at[src],
                dst_ref=xg.at[src],
                send_sem=send_sems.at[0],
                recv_sem=recv_sems.at[src],
                device_id=(src,),
                device_id_type=pl.DeviceIdType.MESH,
            )

        def mk_lcopy():
            return pltpu.make_async_copy(
                x_ref.at[pl.ds(me * M_PER, M_PER), :], xg.at[me], lsem
            )

        def mk_stage(jj, c, slot):
            return pltpu.make_async_copy(
                w_ref.at[pl.ds(src_of(jj) * K_BLK + c * WFR, WFR), :],
                wstage.at[slot],
                wsems.at[slot],
            )

        def mk_fstore(tt):
            sl = lax.rem(tt, 2)
            return pltpu.make_async_copy(
                fstage.at[sl],
                o_ref.at[:, pl.ds(tt * NTW, NTW)],
                fsems.at[sl],
            )

        @pl.when(s == 0)
        def _():
            barrier_sem = pltpu.get_barrier_semaphore()
            for off in (1, 2, 3):
                t = lax.rem(me + off, N_DEV)
                pl.semaphore_signal(
                    barrier_sem, inc=1,
                    device_id=(t,), device_id_type=pl.DeviceIdType.MESH,
                )
            pl.semaphore_wait(barrier_sem, 3)
            mk_send(1, 0).start()
            mk_send(3, 1).start()
            mk_send(2, 2).start()
            mk_lcopy().start()

        @pl.when(nt == 0)
        def _():
            mk_stage(j, 0, 0).start()
            for c in range(WF):
                if c + 1 < WF:
                    mk_stage(j, c + 1, (c + 1) % 2).start()
                mk_stage(j, c, c % 2).wait()
                wblk[c * WFR:(c + 1) * WFR, :] = (
                    wstage[c % 2].astype(jnp.bfloat16)
                )

            @pl.when(j == 0)
            def _():
                mk_lcopy().wait()

            @pl.when(j > 0)
            def _():
                mk_recv(src_of(j)).wait_recv()

            cp = pltpu.make_async_copy(xg.at[src_of(j)], xb, xbsem)
            cp.start()
            cp.wait()

        cols = pl.ds(nt * NTW, NTW)
        val = jnp.dot(
            xb[...], wblk[:, cols], preferred_element_type=jnp.float32
        )

        @pl.when(j == 0)
        def _():
            o16[:, cols] = val.astype(jnp.bfloat16)

        @pl.when((j == 1) | (j == 2))
        def _():
            o16[:, cols] = (
                o16[:, cols].astype(jnp.float32) + val
            ).astype(jnp.bfloat16)

        @pl.when(j == N_DEV - 1)
        def _():
            @pl.when(nt >= 2)
            def _():
                mk_fstore(nt - 2).wait()

            sl = lax.rem(nt, 2)
            fstage[sl] = jnp.maximum(
                o16[:, cols].astype(jnp.float32) + val, 0.0
            )
            mk_fstore(nt).start()

        @pl.when(s == N_STEPS - 1)
        def _():
            mk_fstore(NTS - 2).wait()
            mk_fstore(NTS - 1).wait()
            for off, sl in ((1, 0), (3, 1), (2, 2)):
                mk_send(off, sl).wait_send()

    out, _ = pl.pallas_call(
        body,
        grid=(N_DEV, NTS),
        out_shape=[
            jax.ShapeDtypeStruct((M_PER, N_TOT), jnp.float32),
            jax.ShapeDtypeStruct((N_DEV, M_PER, K_BLK), jnp.bfloat16),
        ],
        in_specs=[
            pl.BlockSpec(memory_space=pl.ANY),
            pl.BlockSpec(memory_space=pl.ANY),
        ],
        out_specs=[
            pl.BlockSpec(memory_space=pl.ANY),
            pl.BlockSpec(memory_space=pl.ANY),
        ],
        scratch_shapes=[
            pltpu.VMEM((M_PER, N_TOT), jnp.bfloat16),
            pltpu.VMEM((K_BLK, N_TOT), jnp.bfloat16),
            pltpu.VMEM((M_PER, K_BLK), jnp.bfloat16),
            pltpu.VMEM((2, WFR, N_TOT), jnp.float32),
            pltpu.VMEM((2, M_PER, NTW), jnp.float32),
            pltpu.SemaphoreType.DMA,
            pltpu.SemaphoreType.DMA,
            pltpu.SemaphoreType.DMA((2,)),
            pltpu.SemaphoreType.DMA((2,)),
            pltpu.SemaphoreType.DMA((3,)),
            pltpu.SemaphoreType.DMA((N_DEV,)),
        ],
        compiler_params=pltpu.CompilerParams(
            collective_id=0,
            vmem_limit_bytes=63 * 1024 * 1024,
        ),
    )(x16, w_mat)
    return out


# device time: 385150 ns/iter; 1.6176x vs baseline; 1.4017x over previous
import jax
import jax.numpy as jnp
from jax import lax
from jax.experimental import pallas as pl
from jax.experimental.pallas import tpu as pltpu

N_DEV = 4
M_PER = 2048
K_BLK = 2048
N_TOT = 4096
NTW = 256
NTS = N_TOT // NTW
WFR = 128
WF = K_BLK // WFR
N_STEPS = N_DEV * NTS


def kernel(x, w_mat):
    x16 = x.astype(jnp.bfloat16)

    def body(x_ref, w_ref, o_ref, o16, wblk, xsl, wstage, fstage,
             lsem, wsems, fsems, send_sems, recv_sems, credit_sems):
        j = pl.program_id(0)
        nt = pl.program_id(1)
        s = j * NTS + nt
        me = lax.axis_index("i")

        def src_of(jj):
            return lax.rem(me + lax.bitwise_xor(jj, jj // 2), N_DEV)

        def mk_send(off, ssem, xslot, rsem):
            t = lax.rem(me + off, N_DEV)
            return pltpu.make_async_remote_copy(
                src_ref=x_ref.at[pl.ds(t * M_PER, M_PER), :],
                dst_ref=xsl.at[xslot],
                send_sem=send_sems.at[ssem],
                recv_sem=recv_sems.at[rsem],
                device_id=(t,),
                device_id_type=pl.DeviceIdType.MESH,
            )

        def mk_recv(xslot, rsem):
            return pltpu.make_async_remote_copy(
                src_ref=xsl.at[xslot],
                dst_ref=xsl.at[xslot],
                send_sem=send_sems.at[0],
                recv_sem=recv_sems.at[rsem],
                device_id=(0,),
                device_id_type=pl.DeviceIdType.MESH,
            )

        def mk_stage(jj, c, slot):
            return pltpu.make_async_copy(
                w_ref.at[pl.ds(src_of(jj) * K_BLK + c * WFR, WFR), :],
                wstage.at[slot],
                wsems.at[slot],
            )

        def mk_fstore(tt):
            sl = lax.rem(tt, 2)
            return pltpu.make_async_copy(
                fstage.at[sl],
                o_ref.at[:, pl.ds(tt * NTW, NTW)],
                fsems.at[sl],
            )

        SENDS = {
            "j1": (3, 0, 1, 1),
            "j2": (1, 1, 0, 2),
            "j3": (2, 2, 1, 3),
        }

        @pl.when(s == 0)
        def _():
            barrier_sem = pltpu.get_barrier_semaphore()
            for off in (1, 2, 3):
                t = lax.rem(me + off, N_DEV)
                pl.semaphore_signal(
                    barrier_sem, inc=1,
                    device_id=(t,), device_id_type=pl.DeviceIdType.MESH,
                )
            pl.semaphore_wait(barrier_sem, 3)
            mk_send(*SENDS["j1"]).start()
            cp = pltpu.make_async_copy(
                x_ref.at[pl.ds(me * M_PER, M_PER), :], xsl.at[0], lsem
            )
            cp.start()

        @pl.when(nt == 0)
        def _():
            mk_stage(j, 0, 0).start()
            for c in range(WF):
                if c + 1 < WF:
                    mk_stage(j, c + 1, (c + 1) % 2).start()
                mk_stage(j, c, c % 2).wait()
                wblk[c * WFR:(c + 1) * WFR, :] = (
                    wstage[c % 2].astype(jnp.bfloat16)
                )

            @pl.when(j == 0)
            def _():
                pltpu.make_async_copy(
                    x_ref.at[pl.ds(me * M_PER, M_PER), :], xsl.at[0], lsem
                ).wait()

            @pl.when(j == 1)
            def _():
                pl.semaphore_signal(
                    credit_sems.at[0], inc=1,
                    device_id=(lax.rem(me + 3, N_DEV),),
                    device_id_type=pl.DeviceIdType.MESH,
                )
                pl.semaphore_wait(credit_sems.at[0], 1)
                mk_send(*SENDS["j2"]).start()
                mk_recv(1, 1).wait_recv()

            @pl.when(j == 2)
            def _():
                pl.semaphore_signal(
                    credit_sems.at[1], inc=1,
                    device_id=(lax.rem(me + 2, N_DEV),),
                    device_id_type=pl.DeviceIdType.MESH,
                )
                pl.semaphore_wait(credit_sems.at[1], 1)
                mk_send(*SENDS["j3"]).start()
                mk_recv(0, 2).wait_recv()

            @pl.when(j == 3)
            def _():
                mk_recv(1, 3).wait_recv()

        cols = pl.ds(nt * NTW, NTW)
        val = jnp.dot(
            xsl[lax.rem(j, 2)], wblk[:, cols],
            preferred_element_type=jnp.float32,
        )

        @pl.when(j == 0)
        def _():
            o16[:, cols] = val.astype(jnp.bfloat16)

        @pl.when((j == 1) | (j == 2))
        def _():
            o16[:, cols] = (
                o16[:, cols].astype(jnp.float32) + val
            ).astype(jnp.bfloat16)

        @pl.when(j == N_DEV - 1)
        def _():
            @pl.when(nt >= 2)
            def _():
                mk_fstore(nt - 2).wait()

            sl = lax.rem(nt, 2)
            fstage[sl] = jnp.maximum(
                o16[:, cols].astype(jnp.float32) + val, 0.0
            )
            mk_fstore(nt).start()

        @pl.when(s == N_STEPS - 1)
        def _():
            mk_fstore(NTS - 2).wait()
            mk_fstore(NTS - 1).wait()
            for key in ("j1", "j2", "j3"):
                mk_send(*SENDS[key]).wait_send()

    return pl.pallas_call(
        body,
        grid=(N_DEV, NTS),
        out_shape=jax.ShapeDtypeStruct((M_PER, N_TOT), jnp.float32),
        in_specs=[
            pl.BlockSpec(memory_space=pl.ANY),
            pl.BlockSpec(memory_space=pl.ANY),
        ],
        out_specs=pl.BlockSpec(memory_space=pl.ANY),
        scratch_shapes=[
            pltpu.VMEM((M_PER, N_TOT), jnp.bfloat16),
            pltpu.VMEM((K_BLK, N_TOT), jnp.bfloat16),
            pltpu.VMEM((2, M_PER, K_BLK), jnp.bfloat16),
            pltpu.VMEM((2, WFR, N_TOT), jnp.float32),
            pltpu.VMEM((2, M_PER, NTW), jnp.float32),
            pltpu.SemaphoreType.DMA,
            pltpu.SemaphoreType.DMA((2,)),
            pltpu.SemaphoreType.DMA((2,)),
            pltpu.SemaphoreType.DMA((3,)),
            pltpu.SemaphoreType.DMA((N_DEV,)),
            pltpu.SemaphoreType.REGULAR((2,)),
        ],
        compiler_params=pltpu.CompilerParams(
            collective_id=0,
            vmem_limit_bytes=63 * 1024 * 1024,
        ),
    )(x16, w_mat)


# device time: 370330 ns/iter; 1.6823x vs baseline; 1.0400x over previous
import jax
import jax.numpy as jnp
from jax import lax
from jax.experimental import pallas as pl
from jax.experimental.pallas import tpu as pltpu

N_DEV = 4
M_PER = 2048
K_BLK = 2048
N_TOT = 4096
NTW = 256
NTS = N_TOT // NTW
WFR = 128
WF = K_BLK // WFR
N_STEPS = N_DEV * NTS


def kernel(x, w_mat):
    x16 = x.astype(jnp.bfloat16)

    def body(x_ref, w_ref, o_ref, o16, wblk, xsl, wstage, fstage,
             lsem, wsems, fsems, send_sems, recv_sems, credit_sems):
        j = pl.program_id(0)
        nt = pl.program_id(1)
        s = j * NTS + nt
        me = lax.axis_index("i")

        def src_of(jj):
            return lax.rem(me + lax.bitwise_xor(jj, jj // 2), N_DEV)

        def mk_send(off, ssem, xslot, rsem):
            t = lax.rem(me + off, N_DEV)
            return pltpu.make_async_remote_copy(
                src_ref=x_ref.at[pl.ds(t * M_PER, M_PER), :],
                dst_ref=xsl.at[xslot],
                send_sem=send_sems.at[ssem],
                recv_sem=recv_sems.at[rsem],
                device_id=(t,),
                device_id_type=pl.DeviceIdType.MESH,
            )

        def mk_recv(xslot, rsem):
            return pltpu.make_async_remote_copy(
                src_ref=xsl.at[xslot],
                dst_ref=xsl.at[xslot],
                send_sem=send_sems.at[0],
                recv_sem=recv_sems.at[rsem],
                device_id=(0,),
                device_id_type=pl.DeviceIdType.MESH,
            )

        def mk_stage(jj, c, slot):
            return pltpu.make_async_copy(
                w_ref.at[pl.ds(src_of(jj) * K_BLK + c * WFR, WFR), :],
                wstage.at[slot],
                wsems.at[slot],
            )

        def mk_fstore(tt):
            sl = lax.rem(tt, 2)
            return pltpu.make_async_copy(
                fstage.at[sl],
                o_ref.at[:, pl.ds(tt * NTW, NTW)],
                fsems.at[sl],
            )

        SENDS = {
            "j1": (3, 0, 1, 1),
            "j2": (1, 1, 0, 2),
            "j3": (2, 2, 1, 3),
        }

        @pl.when(s == 0)
        def _():
            barrier_sem = pltpu.get_barrier_semaphore()
            for off in (1, 2, 3):
                t = lax.rem(me + off, N_DEV)
                pl.semaphore_signal(
                    barrier_sem, inc=1,
                    device_id=(t,), device_id_type=pl.DeviceIdType.MESH,
                )
            pl.semaphore_wait(barrier_sem, 3)
            mk_send(*SENDS["j1"]).start()
            cp = pltpu.make_async_copy(
                x_ref.at[pl.ds(me * M_PER, M_PER), :], xsl.at[0], lsem
            )
            cp.start()

        @pl.when(nt == 0)
        def _():
            @pl.when(j == 1)
            def _():
                pl.semaphore_signal(
                    credit_sems.at[0], inc=1,
                    device_id=(lax.rem(me + 3, N_DEV),),
                    device_id_type=pl.DeviceIdType.MESH,
                )
                pl.semaphore_wait(credit_sems.at[0], 1)
                mk_send(*SENDS["j2"]).start()

            @pl.when(j == 2)
            def _():
                pl.semaphore_signal(
                    credit_sems.at[1], inc=1,
                    device_id=(lax.rem(me + 2, N_DEV),),
                    device_id_type=pl.DeviceIdType.MESH,
                )
                pl.semaphore_wait(credit_sems.at[1], 1)
                mk_send(*SENDS["j3"]).start()

            mk_stage(j, 0, 0).start()
            for c in range(WF):
                if c + 1 < WF:
                    mk_stage(j, c + 1, (c + 1) % 2).start()
                mk_stage(j, c, c % 2).wait()
                wblk[c * WFR:(c + 1) * WFR, :] = (
                    wstage[c % 2].astype(jnp.bfloat16)
                )

            @pl.when(j == 0)
            def _():
                pltpu.make_async_copy(
                    x_ref.at[pl.ds(me * M_PER, M_PER), :], xsl.at[0], lsem
                ).wait()

            @pl.when(j == 1)
            def _():
                mk_recv(1, 1).wait_recv()

            @pl.when(j == 2)
            def _():
                mk_recv(0, 2).wait_recv()

            @pl.when(j == 3)
            def _():
                mk_recv(1, 3).wait_recv()

        cols = pl.ds(nt * NTW, NTW)
        val = jnp.dot(
            xsl[lax.rem(j, 2)], wblk[:, cols],
            preferred_element_type=jnp.float32,
        )

        @pl.when(j == 0)
        def _():
            o16[:, cols] = val.astype(jnp.bfloat16)

        @pl.when((j == 1) | (j == 2))
        def _():
            o16[:, cols] = (
                o16[:, cols].astype(jnp.float32) + val
            ).astype(jnp.bfloat16)

        @pl.when(j == N_DEV - 1)
        def _():
            @pl.when(nt >= 2)
            def _():
                mk_fstore(nt - 2).wait()

            sl = lax.rem(nt, 2)
            fstage[sl] = jnp.maximum(
                o16[:, cols].astype(jnp.float32) + val, 0.0
            )
            mk_fstore(nt).start()

        @pl.when(s == N_STEPS - 1)
        def _():
            mk_fstore(NTS - 2).wait()
            mk_fstore(NTS - 1).wait()
            for key in ("j1", "j2", "j3"):
                mk_send(*SENDS[key]).wait_send()

    return pl.pallas_call(
        body,
        grid=(N_DEV, NTS),
        out_shape=jax.ShapeDtypeStruct((M_PER, N_TOT), jnp.float32),
        in_specs=[
            pl.BlockSpec(memory_space=pl.ANY),
            pl.BlockSpec(memory_space=pl.ANY),
        ],
        out_specs=pl.BlockSpec(memory_space=pl.ANY),
        scratch_shapes=[
            pltpu.VMEM((M_PER, N_TOT), jnp.bfloat16),
            pltpu.VMEM((K_BLK, N_TOT), jnp.bfloat16),
            pltpu.VMEM((2, M_PER, K_BLK), jnp.bfloat16),
            pltpu.VMEM((2, WFR, N_TOT), jnp.float32),
            pltpu.VMEM((2, M_PER, NTW), jnp.float32),
            pltpu.SemaphoreType.DMA,
            pltpu.SemaphoreType.DMA((2,)),
            pltpu.SemaphoreType.DMA((2,)),
            pltpu.SemaphoreType.DMA((3,)),
            pltpu.SemaphoreType.DMA((N_DEV,)),
            pltpu.SemaphoreType.REGULAR((2,)),
        ],
        compiler_params=pltpu.CompilerParams(
            collective_id=0,
            vmem_limit_bytes=63 * 1024 * 1024,
        ),
    )(x16, w_mat)
